# baseline (speedup 1.0000x reference)
import functools

import jax
import jax.numpy as jnp
from jax import lax
from jax.experimental import pallas as pl
from jax.experimental.pallas import tpu as pltpu
from jax.experimental.pallas import tpu_sc as plsc

N = 10000
C = 128
E = 320000
NC, NS = 2, 16
NW = NC * NS
EPW = E // NW
KC = 128
EPWP = 10240
NCH = EPWP // KC
NB = 2
NP = 10016
PHASES = ((0, 24), (24, 24), (48, 24), (72, 8))
RCH = 80
NRCH = N // RCH
NJ = 8
KCC = 100
NCHC = EPW // KCC
CW = 128
RCHC = 100
NRCHC = N // RCHC
NJC = 7

_sc_mesh = plsc.VectorSubcoreMesh(core_axis_name="c", subcore_axis_name="s")


def _silu(x):
    return x * jax.nn.sigmoid(x)


@functools.partial(
    pl.kernel,
    out_type=jax.ShapeDtypeStruct((NC, NRCH, RCH, C), jnp.float32),
    mesh=_sc_mesh,
    scratch_types=[
        pltpu.VMEM((24, KC), jnp.int32),
        pltpu.VMEM((24, KC), jnp.int32),
        [pltpu.VMEM((KC, C), jnp.float32) for _ in range(NB)],
        [pltpu.SemaphoreType.DMA for _ in range(NB)],
        [pltpu.SemaphoreType.DMA for _ in range(NB)],
        pltpu.VMEM_SHARED((N, C), jnp.float32),
    ],
)
def _segsum_sc(table_hbm, src_hbm, dst_hbm, out_hbm,
               src_v, dst_v, rows_v, sem_g, sem_s, acc_sh):
    c = lax.axis_index("c")
    s = lax.axis_index("s")
    wid = s * NC + c

    zeros16 = jnp.zeros((16,), jnp.float32)

    @pl.loop(0, KC)
    def _fill_zero(r):
        for cc in range(C // 16):
            rows_v[0][r, pl.ds(cc * 16, 16)] = zeros16

    for jj in range(NJ):
        k = s + NS * jj

        @pl.when(k < NRCH)
        def _zero_acc():
            pltpu.sync_copy(rows_v[0].at[pl.ds(0, RCH)],
                            acc_sh.at[pl.ds(k * RCH, RCH)])

    plsc.subcore_barrier()

    for row0, nrows in PHASES:
        ngrp = nrows // NB
        pltpu.sync_copy(src_hbm.at[wid, pl.ds(row0, nrows)],
                        src_v.at[pl.ds(0, nrows)])
        pltpu.sync_copy(dst_hbm.at[wid, pl.ds(row0, nrows)],
                        dst_v.at[pl.ds(0, nrows)])

        @pl.loop(0, ngrp)
        def _edge_grp(g):
            ds = []
            for b in range(NB):
                j = NB * g + b

                @pl.when(g > 0)
                def _drain_prev():
                    pltpu.make_async_copy(rows_v[b], acc_sh.at[dst_v.at[j]],
                                          sem_s[b]).wait()

                ds.append(pltpu.async_copy(table_hbm.at[src_v.at[j]],
                                           rows_v[b], sem_g[b]))
            for b in range(NB):
                j = NB * g + b
                ds[b].wait()
                pltpu.async_copy(rows_v[b], acc_sh.at[dst_v.at[j]],
                                 sem_s[b], add=True)

        for b in range(NB):
            pltpu.make_async_copy(rows_v[b], acc_sh.at[dst_v.at[b]],
                                  sem_s[b]).wait()
    plsc.subcore_barrier()

    for jj in range(NJ):
        k = s + NS * jj

        @pl.when(k < NRCH)
        def _drain():
            pltpu.sync_copy(acc_sh.at[pl.ds(k * RCH, RCH)],
                            rows_v[0].at[pl.ds(0, RCH)])
            pltpu.sync_copy(rows_v[0].at[pl.ds(0, RCH)], out_hbm.at[c, k])


@functools.partial(
    pl.kernel,
    out_type=jax.ShapeDtypeStruct((2, NC, NRCHC, RCHC, CW), jnp.float32),
    mesh=_sc_mesh,
    scratch_types=[
        pltpu.VMEM((NCHC, KCC), jnp.int32),
        pltpu.VMEM((KCC, CW), jnp.float32),
        pltpu.VMEM_SHARED((N, CW), jnp.float32),
        pltpu.SemaphoreType.DMA,
    ],
)
def _counts_sc(dst_a_hbm, dst_b_hbm, out_hbm, dst_v, ones_v, acc_sh, sem):
    c = lax.axis_index("c")
    s = lax.axis_index("s")
    wid = s * NC + c

    ones16 = jnp.ones((16,), jnp.float32)
    zeros16 = jnp.zeros((16,), jnp.float32)

    for d, dst_hbm in enumerate((dst_a_hbm, dst_b_hbm)):
        @pl.loop(0, KCC)
        def _fill_zero(r):
            for cc in range(CW // 16):
                ones_v[r, pl.ds(cc * 16, 16)] = zeros16

        for jj in range(NJC):
            k = s + NS * jj

            @pl.when(k < NRCHC)
            def _zero_acc():
                pltpu.sync_copy(ones_v, acc_sh.at[pl.ds(k * RCHC, RCHC)])

        @pl.loop(0, KCC)
        def _fill_ones(r):
            for cc in range(CW // 16):
                ones_v[r, pl.ds(cc * 16, 16)] = ones16

        pltpu.sync_copy(dst_hbm.at[wid], dst_v)
        plsc.subcore_barrier()

        @pl.loop(0, NCHC)
        def _edge_chunk(j):
            pltpu.async_copy(ones_v, acc_sh.at[dst_v.at[j]], sem, add=True)

        @pl.loop(0, NCHC)
        def _drain_chunks(j):
            pltpu.make_async_copy(ones_v, acc_sh.at[dst_v.at[j]], sem).wait()

        plsc.subcore_barrier()

        for jj in range(NJC):
            k = s + NS * jj

            @pl.when(k < NRCHC)
            def _drain():
                pltpu.sync_copy(acc_sh.at[pl.ds(k * RCHC, RCHC)], ones_v)
                pltpu.sync_copy(ones_v, out_hbm.at[d, c, k])


_BR = 1000


def _enc_body(x_ref, w1_ref, b1_ref, w2_ref, b2_ref, o_ref):
    t = _silu(jnp.dot(x_ref[...], w1_ref[...],
                      preferred_element_type=jnp.float32) + b1_ref[...])
    o_ref[...] = jnp.dot(t, w2_ref[...],
                         preferred_element_type=jnp.float32) + b2_ref[...]


def _encoder(x, w1, b1, w2, b2):
    rows = x.shape[0]
    grid = rows // _BR
    full = lambda i: (0, 0)
    return pl.pallas_call(
        _enc_body,
        grid=(grid,),
        in_specs=[
            pl.BlockSpec((_BR, C), lambda i: (i, 0)),
            pl.BlockSpec((C, C), full),
            pl.BlockSpec((1, C), full),
            pl.BlockSpec((C, C), full),
            pl.BlockSpec((1, C), full),
        ],
        out_specs=pl.BlockSpec((_BR, C), lambda i: (i, 0)),
        out_shape=jax.ShapeDtypeStruct((rows, C), jnp.float32),
    )(x, w1, b1, w2, b2)


def _layer_body(s0_ref, s1_ref, inv_ref, h_ref, wm_ref, wu_ref, ws_ref, b_ref,
                o_ref):
    m = (s0_ref[...] + s1_ref[...]) * inv_ref[...]
    t = jnp.dot(m, wm_ref[...], preferred_element_type=jnp.float32)
    o_ref[...] = _silu(
        jnp.dot(t, wu_ref[...], preferred_element_type=jnp.float32)
        + jnp.dot(h_ref[...], ws_ref[...], preferred_element_type=jnp.float32)
        + b_ref[...])


def _layer_update(s0, s1, inv, h, wm, wu, ws, b):
    grid = N // _BR
    full = lambda i: (0, 0)
    return pl.pallas_call(
        _layer_body,
        grid=(grid,),
        in_specs=[
            pl.BlockSpec((_BR, C), lambda i: (i, 0)),
            pl.BlockSpec((_BR, C), lambda i: (i, 0)),
            pl.BlockSpec((_BR, 1), lambda i: (i, 0)),
            pl.BlockSpec((_BR, C), lambda i: (i, 0)),
            pl.BlockSpec((C, C), full),
            pl.BlockSpec((C, C), full),
            pl.BlockSpec((C, C), full),
            pl.BlockSpec((1, C), full),
        ],
        out_specs=pl.BlockSpec((_BR, C), lambda i: (i, 0)),
        out_shape=jax.ShapeDtypeStruct((N, C), jnp.float32),
    )(s0, s1, inv, h, wm, wu, ws, b)


def _head_body(x_ref, w1_ref, b1_ref, w2_ref, b2_ref, o_ref):
    t = _silu(jnp.dot(x_ref[...], w1_ref[...],
                      preferred_element_type=jnp.float32) + b1_ref[...])
    o_ref[...] = jnp.dot(t, w2_ref[...],
                         preferred_element_type=jnp.float32) + b2_ref[...]


def _head(hv, w1, b1, w2p, b2p):
    rows = hv.shape[0]
    grid = rows // _BR
    full = lambda i: (0, 0)
    return pl.pallas_call(
        _head_body,
        grid=(grid,),
        in_specs=[
            pl.BlockSpec((_BR, 2 * C), lambda i: (i, 0)),
            pl.BlockSpec((2 * C, 2 * C), full),
            pl.BlockSpec((1, 2 * C), full),
            pl.BlockSpec((2 * C, C), full),
            pl.BlockSpec((1, C), full),
        ],
        out_specs=pl.BlockSpec((_BR, C), lambda i: (i, 0)),
        out_shape=jax.ShapeDtypeStruct((rows, C), jnp.float32),
    )(hv, w1, b1, w2p, b2p)


def kernel(x_lit, x_cls, edge_index_l2c, edge_index_c2l,
           W_enc1, b_enc1, W_enc2, b_enc2,
           W_msg_l2c, W_upd_cls, W_self_cls, b_cls,
           W_msg_c2l, W_upd_lit, W_self_lit, b_lit,
           W_out1, b_out1, W_out2, b_out2):
    pad_s = jnp.full((NW, EPWP - EPW), N, jnp.int32)
    pad_d = jnp.zeros((NW, EPWP - EPW), jnp.int32)

    def _pad3(idx, pad):
        return jnp.concatenate([idx.reshape(NW, EPW), pad], axis=1
                               ).reshape(NW, NCH, KC)

    src_lc = _pad3(edge_index_l2c[0], pad_s)
    dst_lc = _pad3(edge_index_l2c[1], pad_d)
    src_cl = _pad3(edge_index_c2l[0], pad_s)
    dst_cl = _pad3(edge_index_c2l[1], pad_d)

    def _pad_table(h):
        return jnp.concatenate([h, jnp.zeros((NP - N, C), jnp.float32)], axis=0)

    cnts = _counts_sc(edge_index_l2c[1].reshape(NW, NCHC, KCC),
                      edge_index_c2l[1].reshape(NW, NCHC, KCC)
                      ).reshape(2, NC, N, CW)
    cnt_c = cnts[0, 0, :, :1] + cnts[0, 1, :, :1]
    cnt_l = cnts[1, 0, :, :1] + cnts[1, 1, :, :1]
    inv_c = 1.0 / jnp.maximum(cnt_c, 1.0)
    inv_l = 1.0 / jnp.maximum(cnt_l, 1.0)

    x = jnp.concatenate([x_lit, x_cls], axis=0)
    h = _encoder(x, W_enc1, b_enc1.reshape(1, C), W_enc2, b_enc2.reshape(1, C))
    h_lit, h_cls = h[:N], h[N:]

    L = W_msg_l2c.shape[0]
    for i in range(L):
        S = _segsum_sc(_pad_table(h_lit), src_lc, dst_lc).reshape(NC, N, C)
        h_cls = _layer_update(S[0], S[1], inv_c, h_cls,
                              W_msg_l2c[i], W_upd_cls[i], W_self_cls[i],
                              b_cls[i].reshape(1, C))
        S = _segsum_sc(_pad_table(h_cls), src_cl, dst_cl).reshape(NC, N, C)
        h_lit = _layer_update(S[0], S[1], inv_l, h_lit,
                              W_msg_c2l[i], W_upd_lit[i], W_self_lit[i],
                              b_lit[i].reshape(1, C))

    h_var = h_lit.reshape(N // 2, 2 * C)
    w2p = jnp.zeros((2 * C, C), jnp.float32).at[:, :2].set(W_out2)
    b2p = jnp.zeros((1, C), jnp.float32).at[:, :2].set(b_out2.reshape(1, 2))
    y = _head(h_var, W_out1, b_out1.reshape(1, 2 * C), w2p, b2p)
    return y[:, :2]


# speedup vs baseline: 2.2772x; 2.2772x over previous
import functools

import jax
import jax.numpy as jnp
from jax import lax
from jax.experimental import pallas as pl
from jax.experimental.pallas import tpu as pltpu
from jax.experimental.pallas import tpu_sc as plsc

N = 10000
C = 128
E = 320000
NC, NS = 2, 16
NW = NC * NS
EPW = E // NW
KC = 80
NCH = EPW // KC
NB = 2
PHASES = ((0, 64), (64, 61))
RCH = 80
NRCH = N // RCH
NJ = 8
KCC = 100
NCHC = EPW // KCC
CW = 128
RCHC = 100
NRCHC = N // RCHC
NJC = 7

_sc_mesh = plsc.VectorSubcoreMesh(core_axis_name="c", subcore_axis_name="s")


def _silu(x):
    return x * jax.nn.sigmoid(x)


@functools.partial(
    pl.kernel,
    out_type=jax.ShapeDtypeStruct((NC, NRCH, RCH, C), jnp.float32),
    mesh=_sc_mesh,
    scratch_types=[
        pltpu.VMEM((64, KC), jnp.int32),
        pltpu.VMEM((64, KC), jnp.int32),
        [pltpu.VMEM((KC, C), jnp.float32) for _ in range(NB)],
        [pltpu.SemaphoreType.DMA for _ in range(NB)],
        [pltpu.SemaphoreType.DMA for _ in range(NB)],
        pltpu.VMEM_SHARED((N, C), jnp.float32),
    ],
)
def _segsum_sc(table_hbm, src_hbm, dst_hbm, out_hbm,
               src_v, dst_v, rows_v, sem_g, sem_s, acc_sh):
    c = lax.axis_index("c")
    s = lax.axis_index("s")
    wid = s * NC + c

    zeros16 = jnp.zeros((16,), jnp.float32)

    @pl.loop(0, KC)
    def _fill_zero(r):
        for cc in range(C // 16):
            rows_v[0][r, pl.ds(cc * 16, 16)] = zeros16

    for jj in range(NJ):
        k = s + NS * jj

        @pl.when(k < NRCH)
        def _zero_acc():
            pltpu.sync_copy(rows_v[0].at[pl.ds(0, RCH)],
                            acc_sh.at[pl.ds(k * RCH, RCH)])

    plsc.subcore_barrier()

    for row0, nrows in PHASES:
        ngrp = nrows // NB
        pltpu.sync_copy(src_hbm.at[wid, pl.ds(row0, nrows)],
                        src_v.at[pl.ds(0, nrows)])
        pltpu.sync_copy(dst_hbm.at[wid, pl.ds(row0, nrows)],
                        dst_v.at[pl.ds(0, nrows)])

        @pl.loop(0, ngrp)
        def _edge_grp(g):
            ds = []
            for b in range(NB):
                j = NB * g + b

                @pl.when(g > 0)
                def _drain_prev():
                    pltpu.make_async_copy(rows_v[b], acc_sh.at[dst_v.at[j]],
                                          sem_s[b]).wait()

                ds.append(pltpu.async_copy(table_hbm.at[src_v.at[j]],
                                           rows_v[b], sem_g[b]))
            for b in range(NB):
                j = NB * g + b
                ds[b].wait()
                pltpu.async_copy(rows_v[b], acc_sh.at[dst_v.at[j]],
                                 sem_s[b], add=True)

        for b in range(NB):
            pltpu.make_async_copy(rows_v[b], acc_sh.at[dst_v.at[b]],
                                  sem_s[b]).wait()
        if nrows % NB:
            pltpu.async_copy(table_hbm.at[src_v.at[nrows - 1]], rows_v[0],
                             sem_g[0]).wait()
            pltpu.sync_copy(rows_v[0], acc_sh.at[dst_v.at[nrows - 1]],
                            add=True)
    plsc.subcore_barrier()

    for jj in range(NJ):
        k = s + NS * jj

        @pl.when(k < NRCH)
        def _drain():
            pltpu.sync_copy(acc_sh.at[pl.ds(k * RCH, RCH)],
                            rows_v[0].at[pl.ds(0, RCH)])
            pltpu.sync_copy(rows_v[0].at[pl.ds(0, RCH)], out_hbm.at[c, k])


@functools.partial(
    pl.kernel,
    out_type=jax.ShapeDtypeStruct((2, NC, NRCHC, RCHC, CW), jnp.float32),
    mesh=_sc_mesh,
    scratch_types=[
        pltpu.VMEM((NCHC, KCC), jnp.int32),
        pltpu.VMEM((KCC, CW), jnp.float32),
        pltpu.VMEM_SHARED((N, CW), jnp.float32),
        pltpu.SemaphoreType.DMA,
    ],
)
def _counts_sc(dst_a_hbm, dst_b_hbm, out_hbm, dst_v, ones_v, acc_sh, sem):
    c = lax.axis_index("c")
    s = lax.axis_index("s")
    wid = s * NC + c

    ones16 = jnp.ones((16,), jnp.float32)
    zeros16 = jnp.zeros((16,), jnp.float32)

    for d, dst_hbm in enumerate((dst_a_hbm, dst_b_hbm)):
        @pl.loop(0, KCC)
        def _fill_zero(r):
            for cc in range(CW // 16):
                ones_v[r, pl.ds(cc * 16, 16)] = zeros16

        for jj in range(NJC):
            k = s + NS * jj

            @pl.when(k < NRCHC)
            def _zero_acc():
                pltpu.sync_copy(ones_v, acc_sh.at[pl.ds(k * RCHC, RCHC)])

        @pl.loop(0, KCC)
        def _fill_ones(r):
            for cc in range(CW // 16):
                ones_v[r, pl.ds(cc * 16, 16)] = ones16

        pltpu.sync_copy(dst_hbm.at[wid], dst_v)
        plsc.subcore_barrier()

        @pl.loop(0, NCHC)
        def _edge_chunk(j):
            pltpu.async_copy(ones_v, acc_sh.at[dst_v.at[j]], sem, add=True)

        @pl.loop(0, NCHC)
        def _drain_chunks(j):
            pltpu.make_async_copy(ones_v, acc_sh.at[dst_v.at[j]], sem).wait()

        plsc.subcore_barrier()

        for jj in range(NJC):
            k = s + NS * jj

            @pl.when(k < NRCHC)
            def _drain():
                pltpu.sync_copy(acc_sh.at[pl.ds(k * RCHC, RCHC)], ones_v)
                pltpu.sync_copy(ones_v, out_hbm.at[d, c, k])


_BR = 1000


def _enc_body(x_ref, w1_ref, b1_ref, w2_ref, b2_ref, o_ref):
    t = _silu(jnp.dot(x_ref[...], w1_ref[...],
                      preferred_element_type=jnp.float32) + b1_ref[...])
    o_ref[...] = jnp.dot(t, w2_ref[...],
                         preferred_element_type=jnp.float32) + b2_ref[...]


def _encoder(x, w1, b1, w2, b2):
    rows = x.shape[0]
    grid = rows // _BR
    full = lambda i: (0, 0)
    return pl.pallas_call(
        _enc_body,
        grid=(grid,),
        in_specs=[
            pl.BlockSpec((_BR, C), lambda i: (i, 0)),
            pl.BlockSpec((C, C), full),
            pl.BlockSpec((1, C), full),
            pl.BlockSpec((C, C), full),
            pl.BlockSpec((1, C), full),
        ],
        out_specs=pl.BlockSpec((_BR, C), lambda i: (i, 0)),
        out_shape=jax.ShapeDtypeStruct((rows, C), jnp.float32),
    )(x, w1, b1, w2, b2)


def _layer_body(s0_ref, s1_ref, inv_ref, h_ref, wm_ref, wu_ref, ws_ref, b_ref,
                o_ref):
    m = (s0_ref[...] + s1_ref[...]) * inv_ref[...]
    t = jnp.dot(m, wm_ref[...], preferred_element_type=jnp.float32)
    o_ref[...] = _silu(
        jnp.dot(t, wu_ref[...], preferred_element_type=jnp.float32)
        + jnp.dot(h_ref[...], ws_ref[...], preferred_element_type=jnp.float32)
        + b_ref[...])


def _layer_update(s0, s1, inv, h, wm, wu, ws, b):
    grid = N // _BR
    full = lambda i: (0, 0)
    return pl.pallas_call(
        _layer_body,
        grid=(grid,),
        in_specs=[
            pl.BlockSpec((_BR, C), lambda i: (i, 0)),
            pl.BlockSpec((_BR, C), lambda i: (i, 0)),
            pl.BlockSpec((_BR, 1), lambda i: (i, 0)),
            pl.BlockSpec((_BR, C), lambda i: (i, 0)),
            pl.BlockSpec((C, C), full),
            pl.BlockSpec((C, C), full),
            pl.BlockSpec((C, C), full),
            pl.BlockSpec((1, C), full),
        ],
        out_specs=pl.BlockSpec((_BR, C), lambda i: (i, 0)),
        out_shape=jax.ShapeDtypeStruct((N, C), jnp.float32),
    )(s0, s1, inv, h, wm, wu, ws, b)


def _head_body(x_ref, w1_ref, b1_ref, w2_ref, b2_ref, o_ref):
    t = _silu(jnp.dot(x_ref[...], w1_ref[...],
                      preferred_element_type=jnp.float32) + b1_ref[...])
    o_ref[...] = jnp.dot(t, w2_ref[...],
                         preferred_element_type=jnp.float32) + b2_ref[...]


def _head(hv, w1, b1, w2p, b2p):
    rows = hv.shape[0]
    grid = rows // _BR
    full = lambda i: (0, 0)
    return pl.pallas_call(
        _head_body,
        grid=(grid,),
        in_specs=[
            pl.BlockSpec((_BR, 2 * C), lambda i: (i, 0)),
            pl.BlockSpec((2 * C, 2 * C), full),
            pl.BlockSpec((1, 2 * C), full),
            pl.BlockSpec((2 * C, C), full),
            pl.BlockSpec((1, C), full),
        ],
        out_specs=pl.BlockSpec((_BR, C), lambda i: (i, 0)),
        out_shape=jax.ShapeDtypeStruct((rows, C), jnp.float32),
    )(hv, w1, b1, w2p, b2p)


def kernel(x_lit, x_cls, edge_index_l2c, edge_index_c2l,
           W_enc1, b_enc1, W_enc2, b_enc2,
           W_msg_l2c, W_upd_cls, W_self_cls, b_cls,
           W_msg_c2l, W_upd_lit, W_self_lit, b_lit,
           W_out1, b_out1, W_out2, b_out2):
    src_lc = edge_index_l2c[0].reshape(NW, NCH, KC)
    dst_lc = edge_index_l2c[1].reshape(NW, NCH, KC)
    src_cl = edge_index_c2l[0].reshape(NW, NCH, KC)
    dst_cl = edge_index_c2l[1].reshape(NW, NCH, KC)

    cnts = _counts_sc(edge_index_l2c[1].reshape(NW, NCHC, KCC),
                      edge_index_c2l[1].reshape(NW, NCHC, KCC)
                      ).reshape(2, NC, N, CW)
    cnt_c = cnts[0, 0, :, :1] + cnts[0, 1, :, :1]
    cnt_l = cnts[1, 0, :, :1] + cnts[1, 1, :, :1]
    inv_c = 1.0 / jnp.maximum(cnt_c, 1.0)
    inv_l = 1.0 / jnp.maximum(cnt_l, 1.0)

    x = jnp.concatenate([x_lit, x_cls], axis=0)
    h = _encoder(x, W_enc1, b_enc1.reshape(1, C), W_enc2, b_enc2.reshape(1, C))
    h_lit, h_cls = h[:N], h[N:]

    L = W_msg_l2c.shape[0]
    for i in range(L):
        S = _segsum_sc(h_lit, src_lc, dst_lc).reshape(NC, N, C)
        h_cls = _layer_update(S[0], S[1], inv_c, h_cls,
                              W_msg_l2c[i], W_upd_cls[i], W_self_cls[i],
                              b_cls[i].reshape(1, C))
        S = _segsum_sc(h_cls, src_cl, dst_cl).reshape(NC, N, C)
        h_lit = _layer_update(S[0], S[1], inv_l, h_lit,
                              W_msg_c2l[i], W_upd_lit[i], W_self_lit[i],
                              b_lit[i].reshape(1, C))

    h_var = h_lit.reshape(N // 2, 2 * C)
    w2p = jnp.zeros((2 * C, C), jnp.float32).at[:, :2].set(W_out2)
    b2p = jnp.zeros((1, C), jnp.float32).at[:, :2].set(b_out2.reshape(1, 2))
    y = _head(h_var, W_out1, b_out1.reshape(1, 2 * C), w2p, b2p)
    return y[:, :2]


# speedup vs baseline: 2.4321x; 1.0680x over previous
import functools

import jax
import jax.numpy as jnp
from jax import lax
from jax.experimental import pallas as pl
from jax.experimental.pallas import tpu as pltpu
from jax.experimental.pallas import tpu_sc as plsc

N = 10000
C = 128
E = 320000
NC, NS = 2, 16
NW = NC * NS
EPW = E // NW
KC = 100
NCH = EPW // KC
NB = 2
PHASES = ((0, 48), (48, 52))
RCH = 80
NRCH = N // RCH
NJ = 8
KCC = 100
NCHC = EPW // KCC
CW = 128
RCHC = 100
NRCHC = N // RCHC
NJC = 7

_sc_mesh = plsc.VectorSubcoreMesh(core_axis_name="c", subcore_axis_name="s")


def _silu(x):
    return x * jax.nn.sigmoid(x)


@functools.partial(
    pl.kernel,
    out_type=jax.ShapeDtypeStruct((NC, NRCH, RCH, C), jnp.float32),
    mesh=_sc_mesh,
    scratch_types=[
        pltpu.VMEM((52, KC), jnp.int32),
        pltpu.VMEM((52, KC), jnp.int32),
        [pltpu.VMEM((KC, C), jnp.float32) for _ in range(NB)],
        [pltpu.SemaphoreType.DMA for _ in range(NB)],
        [pltpu.SemaphoreType.DMA for _ in range(NB)],
        pltpu.VMEM_SHARED((N, C), jnp.float32),
    ],
)
def _segsum_sc(table_hbm, src_hbm, dst_hbm, out_hbm,
               src_v, dst_v, rows_v, sem_g, sem_s, acc_sh):
    c = lax.axis_index("c")
    s = lax.axis_index("s")
    wid = s * NC + c

    zeros16 = jnp.zeros((16,), jnp.float32)

    @pl.loop(0, KC)
    def _fill_zero(r):
        for cc in range(C // 16):
            rows_v[0][r, pl.ds(cc * 16, 16)] = zeros16

    for jj in range(NJ):
        k = s + NS * jj

        @pl.when(k < NRCH)
        def _zero_acc():
            pltpu.sync_copy(rows_v[0].at[pl.ds(0, RCH)],
                            acc_sh.at[pl.ds(k * RCH, RCH)])

    plsc.subcore_barrier()

    for row0, nrows in PHASES:
        ngrp = nrows // NB
        pltpu.sync_copy(src_hbm.at[wid, pl.ds(row0, nrows)],
                        src_v.at[pl.ds(0, nrows)])
        pltpu.sync_copy(dst_hbm.at[wid, pl.ds(row0, nrows)],
                        dst_v.at[pl.ds(0, nrows)])

        @pl.loop(0, ngrp)
        def _edge_grp(g):
            ds = []
            for b in range(NB):
                j = NB * g + b

                @pl.when(g > 0)
                def _drain_prev():
                    pltpu.make_async_copy(rows_v[b], acc_sh.at[dst_v.at[j]],
                                          sem_s[b]).wait()

                ds.append(pltpu.async_copy(table_hbm.at[src_v.at[j]],
                                           rows_v[b], sem_g[b]))
            for b in range(NB):
                j = NB * g + b
                ds[b].wait()
                pltpu.async_copy(rows_v[b], acc_sh.at[dst_v.at[j]],
                                 sem_s[b], add=True)

        for b in range(NB):
            pltpu.make_async_copy(rows_v[b], acc_sh.at[dst_v.at[b]],
                                  sem_s[b]).wait()
        if nrows % NB:
            pltpu.async_copy(table_hbm.at[src_v.at[nrows - 1]], rows_v[0],
                             sem_g[0]).wait()
            pltpu.sync_copy(rows_v[0], acc_sh.at[dst_v.at[nrows - 1]],
                            add=True)
    plsc.subcore_barrier()

    for jj in range(NJ):
        k = s + NS * jj

        @pl.when(k < NRCH)
        def _drain():
            pltpu.sync_copy(acc_sh.at[pl.ds(k * RCH, RCH)],
                            rows_v[0].at[pl.ds(0, RCH)])
            pltpu.sync_copy(rows_v[0].at[pl.ds(0, RCH)], out_hbm.at[c, k])


@functools.partial(
    pl.kernel,
    out_type=jax.ShapeDtypeStruct((2, NC, NRCHC, RCHC, CW), jnp.float32),
    mesh=_sc_mesh,
    scratch_types=[
        pltpu.VMEM((NCHC, KCC), jnp.int32),
        pltpu.VMEM((KCC, CW), jnp.float32),
        pltpu.VMEM_SHARED((N, CW), jnp.float32),
        pltpu.SemaphoreType.DMA,
    ],
)
def _counts_sc(dst_a_hbm, dst_b_hbm, out_hbm, dst_v, ones_v, acc_sh, sem):
    c = lax.axis_index("c")
    s = lax.axis_index("s")
    wid = s * NC + c

    ones16 = jnp.ones((16,), jnp.float32)
    zeros16 = jnp.zeros((16,), jnp.float32)

    for d, dst_hbm in enumerate((dst_a_hbm, dst_b_hbm)):
        @pl.loop(0, KCC)
        def _fill_zero(r):
            for cc in range(CW // 16):
                ones_v[r, pl.ds(cc * 16, 16)] = zeros16

        for jj in range(NJC):
            k = s + NS * jj

            @pl.when(k < NRCHC)
            def _zero_acc():
                pltpu.sync_copy(ones_v, acc_sh.at[pl.ds(k * RCHC, RCHC)])

        @pl.loop(0, KCC)
        def _fill_ones(r):
            for cc in range(CW // 16):
                ones_v[r, pl.ds(cc * 16, 16)] = ones16

        pltpu.sync_copy(dst_hbm.at[wid], dst_v)
        plsc.subcore_barrier()

        @pl.loop(0, NCHC)
        def _edge_chunk(j):
            pltpu.async_copy(ones_v, acc_sh.at[dst_v.at[j]], sem, add=True)

        @pl.loop(0, NCHC)
        def _drain_chunks(j):
            pltpu.make_async_copy(ones_v, acc_sh.at[dst_v.at[j]], sem).wait()

        plsc.subcore_barrier()

        for jj in range(NJC):
            k = s + NS * jj

            @pl.when(k < NRCHC)
            def _drain():
                pltpu.sync_copy(acc_sh.at[pl.ds(k * RCHC, RCHC)], ones_v)
                pltpu.sync_copy(ones_v, out_hbm.at[d, c, k])


_BR = 1000


def _enc_body(x_ref, w1_ref, b1_ref, w2_ref, b2_ref, o_ref):
    t = _silu(jnp.dot(x_ref[...], w1_ref[...],
                      preferred_element_type=jnp.float32) + b1_ref[...])
    o_ref[...] = jnp.dot(t, w2_ref[...],
                         preferred_element_type=jnp.float32) + b2_ref[...]


def _encoder(x, w1, b1, w2, b2):
    rows = x.shape[0]
    grid = rows // _BR
    full = lambda i: (0, 0)
    return pl.pallas_call(
        _enc_body,
        grid=(grid,),
        in_specs=[
            pl.BlockSpec((_BR, C), lambda i: (i, 0)),
            pl.BlockSpec((C, C), full),
            pl.BlockSpec((1, C), full),
            pl.BlockSpec((C, C), full),
            pl.BlockSpec((1, C), full),
        ],
        out_specs=pl.BlockSpec((_BR, C), lambda i: (i, 0)),
        out_shape=jax.ShapeDtypeStruct((rows, C), jnp.float32),
    )(x, w1, b1, w2, b2)


def _layer_body(s0_ref, s1_ref, inv_ref, h_ref, wm_ref, wu_ref, ws_ref, b_ref,
                o_ref):
    m = (s0_ref[...] + s1_ref[...]) * inv_ref[...]
    t = jnp.dot(m, wm_ref[...], preferred_element_type=jnp.float32)
    o_ref[...] = _silu(
        jnp.dot(t, wu_ref[...], preferred_element_type=jnp.float32)
        + jnp.dot(h_ref[...], ws_ref[...], preferred_element_type=jnp.float32)
        + b_ref[...])


def _layer_update(s0, s1, inv, h, wm, wu, ws, b):
    grid = N // _BR
    full = lambda i: (0, 0)
    return pl.pallas_call(
        _layer_body,
        grid=(grid,),
        in_specs=[
            pl.BlockSpec((_BR, C), lambda i: (i, 0)),
            pl.BlockSpec((_BR, C), lambda i: (i, 0)),
            pl.BlockSpec((_BR, 1), lambda i: (i, 0)),
            pl.BlockSpec((_BR, C), lambda i: (i, 0)),
            pl.BlockSpec((C, C), full),
            pl.BlockSpec((C, C), full),
            pl.BlockSpec((C, C), full),
            pl.BlockSpec((1, C), full),
        ],
        out_specs=pl.BlockSpec((_BR, C), lambda i: (i, 0)),
        out_shape=jax.ShapeDtypeStruct((N, C), jnp.float32),
    )(s0, s1, inv, h, wm, wu, ws, b)


def _head_body(x_ref, w1_ref, b1_ref, w2_ref, b2_ref, o_ref):
    t = _silu(jnp.dot(x_ref[...], w1_ref[...],
                      preferred_element_type=jnp.float32) + b1_ref[...])
    o_ref[...] = jnp.dot(t, w2_ref[...],
                         preferred_element_type=jnp.float32) + b2_ref[...]


def _head(hv, w1, b1, w2p, b2p):
    rows = hv.shape[0]
    grid = rows // _BR
    full = lambda i: (0, 0)
    return pl.pallas_call(
        _head_body,
        grid=(grid,),
        in_specs=[
            pl.BlockSpec((_BR, 2 * C), lambda i: (i, 0)),
            pl.BlockSpec((2 * C, 2 * C), full),
            pl.BlockSpec((1, 2 * C), full),
            pl.BlockSpec((2 * C, C), full),
            pl.BlockSpec((1, C), full),
        ],
        out_specs=pl.BlockSpec((_BR, C), lambda i: (i, 0)),
        out_shape=jax.ShapeDtypeStruct((rows, C), jnp.float32),
    )(hv, w1, b1, w2p, b2p)


def kernel(x_lit, x_cls, edge_index_l2c, edge_index_c2l,
           W_enc1, b_enc1, W_enc2, b_enc2,
           W_msg_l2c, W_upd_cls, W_self_cls, b_cls,
           W_msg_c2l, W_upd_lit, W_self_lit, b_lit,
           W_out1, b_out1, W_out2, b_out2):
    src_lc = edge_index_l2c[0].reshape(NW, NCH, KC)
    dst_lc = edge_index_l2c[1].reshape(NW, NCH, KC)
    src_cl = edge_index_c2l[0].reshape(NW, NCH, KC)
    dst_cl = edge_index_c2l[1].reshape(NW, NCH, KC)

    cnts = _counts_sc(edge_index_l2c[1].reshape(NW, NCHC, KCC),
                      edge_index_c2l[1].reshape(NW, NCHC, KCC)
                      ).reshape(2, NC, N, CW)
    cnt_c = cnts[0, 0, :, :1] + cnts[0, 1, :, :1]
    cnt_l = cnts[1, 0, :, :1] + cnts[1, 1, :, :1]
    inv_c = 1.0 / jnp.maximum(cnt_c, 1.0)
    inv_l = 1.0 / jnp.maximum(cnt_l, 1.0)

    x = jnp.concatenate([x_lit, x_cls], axis=0)
    h = _encoder(x, W_enc1, b_enc1.reshape(1, C), W_enc2, b_enc2.reshape(1, C))
    h_lit, h_cls = h[:N], h[N:]

    L = W_msg_l2c.shape[0]
    for i in range(L):
        S = _segsum_sc(h_lit, src_lc, dst_lc).reshape(NC, N, C)
        h_cls = _layer_update(S[0], S[1], inv_c, h_cls,
                              W_msg_l2c[i], W_upd_cls[i], W_self_cls[i],
                              b_cls[i].reshape(1, C))
        S = _segsum_sc(h_cls, src_cl, dst_cl).reshape(NC, N, C)
        h_lit = _layer_update(S[0], S[1], inv_l, h_lit,
                              W_msg_c2l[i], W_upd_lit[i], W_self_lit[i],
                              b_lit[i].reshape(1, C))

    h_var = h_lit.reshape(N // 2, 2 * C)
    w2p = jnp.zeros((2 * C, C), jnp.float32).at[:, :2].set(W_out2)
    b2p = jnp.zeros((1, C), jnp.float32).at[:, :2].set(b_out2.reshape(1, 2))
    y = _head(h_var, W_out1, b_out1.reshape(1, 2 * C), w2p, b2p)
    return y[:, :2]
